# baseline (speedup 1.0000x reference)
import functools

import jax
import jax.numpy as jnp
from jax import lax
from jax.experimental import pallas as pl
from jax.experimental.pallas import tpu as pltpu
from jax.experimental.pallas import tpu_sc as plsc

N_CORES = 2
N_SUBCORES = 16
LANES = 16
N_WORKERS = N_CORES * N_SUBCORES
BATCH = 16384
FACTORS = 16
BPW = BATCH // N_WORKERS
GROUPS = BPW // LANES


def _mf_body(user_table, movie_table, bias_user, bias_movie, bias,
             user, movie, out,
             uidx_v, midx_v, urows_v, mrows_v, bu_v, bm_v, bias_v, out_v,
             sem):
    wid = lax.axis_index("s") * N_CORES + lax.axis_index("c")
    base = wid * BPW

    pltpu.sync_copy(user.at[pl.ds(base, BPW)], uidx_v)
    pltpu.sync_copy(movie.at[pl.ds(base, BPW)], midx_v)
    pltpu.sync_copy(bias, bias_v)

    cu = pltpu.async_copy(user_table.at[uidx_v], urows_v, sem)
    cm = pltpu.async_copy(movie_table.at[midx_v], mrows_v, sem)
    cbu = pltpu.async_copy(bias_user.at[uidx_v], bu_v, sem)
    cbm = pltpu.async_copy(bias_movie.at[midx_v], bm_v, sem)
    cu.wait()
    cm.wait()
    cbu.wait()
    cbm.wait()

    iota = lax.iota(jnp.int32, LANES)
    bias_vec = bias_v[...]

    def group(g, carry):
        rows = g * LANES + iota
        acc = plsc.load_gather(bu_v, [rows]) + plsc.load_gather(bm_v, [rows])
        acc = acc + bias_vec
        for f in range(FACTORS):
            cols = (iota + f) & (LANES - 1)
            uu = plsc.load_gather(urows_v, [rows, cols])
            mm = plsc.load_gather(mrows_v, [rows, cols])
            acc = acc + uu * mm
        plsc.store_scatter(out_v, [rows], acc)
        return carry

    lax.fori_loop(0, GROUPS, group, 0)

    pltpu.sync_copy(out_v, out.at[pl.ds(base, BPW)])


@functools.partial(jax.jit, static_argnames=())
def _mf(user_table, movie_table, bias_user, bias_movie, bias, user, movie):
    run = functools.partial(
        pl.kernel,
        mesh=plsc.VectorSubcoreMesh(core_axis_name="c", subcore_axis_name="s"),
        out_type=jax.ShapeDtypeStruct((BATCH,), jnp.float32),
        scratch_types=[
            pltpu.VMEM((BPW,), jnp.int32),
            pltpu.VMEM((BPW,), jnp.int32),
            pltpu.VMEM((BPW, FACTORS), jnp.float32),
            pltpu.VMEM((BPW, FACTORS), jnp.float32),
            pltpu.VMEM((BPW,), jnp.float32),
            pltpu.VMEM((BPW,), jnp.float32),
            pltpu.VMEM((LANES,), jnp.float32),
            pltpu.VMEM((BPW,), jnp.float32),
            pltpu.SemaphoreType.DMA,
        ],
        compiler_params=pltpu.CompilerParams(
            needs_layout_passes=False, use_tc_tiling_on_sc=False
        ),
    )(_mf_body)
    return run(user_table, movie_table, bias_user, bias_movie, bias,
               user, movie)


def kernel(user_table, movie_table, bias_user, bias_movie, bias, user, movie):
    return _mf(
        user_table,
        movie_table,
        bias_user.reshape(-1),
        bias_movie.reshape(-1),
        jnp.broadcast_to(bias, (LANES,)),
        user.astype(jnp.int32),
        movie.astype(jnp.int32),
    )


# speedup vs baseline: 3.7063x; 3.7063x over previous
import functools

import jax
import jax.numpy as jnp
from jax import lax
from jax.experimental import pallas as pl
from jax.experimental.pallas import tpu as pltpu
from jax.experimental.pallas import tpu_sc as plsc

N_CORES = 2
N_SUBCORES = 16
LANES = 16
N_WORKERS = N_CORES * N_SUBCORES
BATCH = 16384
FACTORS = 16
BPW = BATCH // N_WORKERS
GROUP = 16
GROUPS = BPW // GROUP


def _mf_body(user_table_t, movie_table_t, bias_user, bias_movie, bias,
             user, movie, out,
             uidx_v, midx_v, ublk_v, mblk_v,
             bu_v, bm_v, bias_v, out_v, sem, bsem):
    wid = lax.axis_index("s") * N_CORES + lax.axis_index("c")
    base = wid * BPW

    pltpu.sync_copy(user.at[pl.ds(base, BPW)], uidx_v)
    pltpu.sync_copy(movie.at[pl.ds(base, BPW)], midx_v)
    pltpu.sync_copy(bias, bias_v)

    bcopies = [
        pltpu.async_copy(bias_user.at[uidx_v], bu_v, bsem),
        pltpu.async_copy(bias_movie.at[midx_v], bm_v, bsem),
    ]

    bias_vec = bias_v[...]
    iota = lax.iota(jnp.int32, LANES)

    def group(g, carry):
        sl = pl.ds(g * GROUP, LANES)
        ridx_u = uidx_v[sl]
        ridx_m = midx_v[sl]
        tile_u = (ridx_u >> 7) * 128
        tile_m = (ridx_m >> 7) * 128
        copies = []
        for j in range(GROUP):
            ou = pl.multiple_of(tile_u[j], 128)
            om = pl.multiple_of(tile_m[j], 128)
            copies.append(pltpu.async_copy(
                user_table_t.at[:, pl.ds(ou, 128)], ublk_v.at[j], sem))
            copies.append(pltpu.async_copy(
                movie_table_t.at[:, pl.ds(om, 128)], mblk_v.at[j], sem))
        for c in copies:
            c.wait()

        lanes_u = ridx_u & 127
        lanes_m = ridx_m & 127
        acc = bias_vec
        for f in range(FACTORS):
            fvec = (iota & 0) + f
            uu = plsc.load_gather(ublk_v, [iota, fvec, lanes_u])
            mm = plsc.load_gather(mblk_v, [iota, fvec, lanes_m])
            acc = acc + uu * mm
        out_v[sl] = acc
        return carry

    lax.fori_loop(0, GROUPS, group, 0)

    for c in bcopies:
        c.wait()

    def addbias(g, carry):
        sl = pl.ds(g * GROUP, LANES)
        out_v[sl] = out_v[sl] + bu_v[sl] + bm_v[sl]
        return carry

    lax.fori_loop(0, GROUPS, addbias, 0)

    pltpu.sync_copy(out_v, out.at[pl.ds(base, BPW)])


@jax.jit
def _mf(user_table_t, movie_table_t, bias_user, bias_movie, bias,
        user, movie):
    run = functools.partial(
        pl.kernel,
        mesh=plsc.VectorSubcoreMesh(core_axis_name="c", subcore_axis_name="s"),
        out_type=jax.ShapeDtypeStruct((BATCH,), jnp.float32),
        scratch_types=[
            pltpu.VMEM((BPW,), jnp.int32),
            pltpu.VMEM((BPW,), jnp.int32),
            pltpu.VMEM((GROUP, FACTORS, 128), jnp.float32),
            pltpu.VMEM((GROUP, FACTORS, 128), jnp.float32),
            pltpu.VMEM((BPW,), jnp.float32),
            pltpu.VMEM((BPW,), jnp.float32),
            pltpu.VMEM((LANES,), jnp.float32),
            pltpu.VMEM((BPW,), jnp.float32),
            pltpu.SemaphoreType.DMA,
            pltpu.SemaphoreType.DMA,
        ],
        compiler_params=pltpu.CompilerParams(needs_layout_passes=False),
    )(_mf_body)
    return run(user_table_t, movie_table_t, bias_user, bias_movie, bias,
               user, movie)


def kernel(user_table, movie_table, bias_user, bias_movie, bias, user, movie):
    return _mf(
        user_table.T,
        movie_table.T,
        bias_user.reshape(-1),
        bias_movie.reshape(-1),
        jnp.broadcast_to(bias, (LANES,)),
        user.astype(jnp.int32),
        movie.astype(jnp.int32),
    )
